# baseline (speedup 1.0000x reference)
import functools
import math

import jax
import jax.numpy as jnp
from jax import lax
from jax.experimental import pallas as pl
from jax.experimental.pallas import tpu as pltpu
from jax.experimental.pallas import tpu_sc as plsc

D_MODEL = 64
SCALE = math.sqrt(D_MODEL)
CHUNK = 128


@functools.cache
def _build(B, V):
    info = plsc.get_sparse_core_info()
    nc, ns, lanes = info.num_cores, info.num_subcores, info.num_lanes
    nw = nc * ns
    b_per_w = B // nw
    n_chunks = b_per_w // CHUNK
    mesh = plsc.VectorSubcoreMesh(core_axis_name="c", subcore_axis_name="s")

    @functools.partial(
        pl.kernel,
        mesh=mesh,
        compiler_params=pltpu.CompilerParams(use_tc_tiling_on_sc=False),
        out_type=jax.ShapeDtypeStruct((B, D_MODEL), jnp.float32),
        scratch_types=[
            pltpu.VMEM((n_chunks, CHUNK), jnp.int32),
            pltpu.VMEM((CHUNK, D_MODEL), jnp.float32),
            pltpu.SemaphoreType.DMA,
        ],
    )
    def emb_kernel(x_hbm, lut_hbm, out_hbm, idx_v, buf, sem):
        wid = lax.axis_index("s") * nc + lax.axis_index("c")
        pltpu.sync_copy(x_hbm.at[pl.ds(wid * n_chunks, n_chunks)], idx_v)

        def chunk_body(g, _):
            pltpu.async_copy(lut_hbm.at[idx_v.at[g]], buf, sem).wait()

            def row_body(r, _):
                for j in range(D_MODEL // lanes):
                    sl = pl.ds(j * lanes, lanes)
                    buf[r, sl] = buf[r, sl] * SCALE
                return ()

            lax.fori_loop(0, CHUNK, row_body, ())
            out_base = wid * b_per_w + g * CHUNK
            pltpu.sync_copy(buf, out_hbm.at[pl.ds(out_base, CHUNK)])
            return ()

        lax.fori_loop(0, n_chunks, chunk_body, ())

    return emb_kernel


def kernel(x, lut):
    B = x.size
    x_2d = x.reshape(B // CHUNK, CHUNK).astype(jnp.int32)
    out = _build(B, lut.shape[0])(x_2d, lut)
    return out.reshape(*x.shape, D_MODEL)


# speedup vs baseline: 1.1861x; 1.1861x over previous
import functools
import math

import jax
import jax.numpy as jnp
from jax import lax
from jax.experimental import pallas as pl
from jax.experimental.pallas import tpu as pltpu
from jax.experimental.pallas import tpu_sc as plsc

D_MODEL = 64
SCALE = math.sqrt(D_MODEL)
CHUNK = 128


@functools.cache
def _build(B, V):
    info = plsc.get_sparse_core_info()
    nc, ns, lanes = info.num_cores, info.num_subcores, info.num_lanes
    nw = nc * ns
    b_per_w = B // nw
    n_chunks = b_per_w // CHUNK
    assert n_chunks % 2 == 0 and n_chunks >= 4
    mesh = plsc.VectorSubcoreMesh(core_axis_name="c", subcore_axis_name="s")

    @functools.partial(
        pl.kernel,
        mesh=mesh,
        compiler_params=pltpu.CompilerParams(use_tc_tiling_on_sc=False),
        out_type=jax.ShapeDtypeStruct((B, D_MODEL), jnp.float32),
        scratch_types=[
            pltpu.VMEM((n_chunks, CHUNK), jnp.int32),
            pltpu.VMEM((CHUNK, D_MODEL), jnp.float32),
            pltpu.VMEM((CHUNK, D_MODEL), jnp.float32),
            pltpu.VMEM((CHUNK, D_MODEL), jnp.float32),
            pltpu.VMEM((CHUNK, D_MODEL), jnp.float32),
            pltpu.SemaphoreType.DMA,
            pltpu.SemaphoreType.DMA,
            pltpu.SemaphoreType.DMA,
            pltpu.SemaphoreType.DMA,
        ],
    )
    def emb_kernel(x_hbm, lut_hbm, out_hbm, idx_v, gbuf0, gbuf1, wbuf0,
                   wbuf1, gsem0, gsem1, wsem0, wsem1):
        wid = lax.axis_index("s") * nc + lax.axis_index("c")
        out_base = wid * b_per_w
        pltpu.sync_copy(x_hbm.at[pl.ds(wid * n_chunks, n_chunks)], idx_v)

        gbufs = (gbuf0, gbuf1)
        gsems = (gsem0, gsem1)
        wbufs = (wbuf0, wbuf1)
        wsems = (wsem0, wsem1)

        def gather_start(g, b):
            pltpu.async_copy(lut_hbm.at[idx_v.at[g]], gbufs[b], gsems[b])

        def scale(b):
            src, dst = gbufs[b], wbufs[b]

            @plsc.parallel_loop(0, CHUNK, unroll=8)
            def _(r):
                for j in range(D_MODEL // lanes):
                    sl = pl.ds(j * lanes, lanes)
                    dst[r, sl] = src[r, sl] * SCALE

        def write_start(g, b):
            pltpu.async_copy(wbufs[b], out_hbm.at[pl.ds(out_base + g * CHUNK, CHUNK)],
                             wsems[b])

        def gather_wait(b):
            pltpu.make_async_copy(lut_hbm.at[idx_v.at[0]], gbufs[b],
                                  gsems[b]).wait()

        def write_wait(b):
            pltpu.make_async_copy(wbufs[b], out_hbm.at[pl.ds(0, CHUNK)],
                                  wsems[b]).wait()

        gather_start(0, 0)
        gather_start(1, 1)
        gather_wait(0)
        scale(0)
        write_start(0, 0)
        gather_start(2, 0)
        gather_wait(1)
        scale(1)
        write_start(1, 1)

        def pair_body(t, _):
            g0 = 2 * t
            gather_start(g0 + 1, 1)
            gather_wait(0)
            write_wait(0)
            scale(0)
            write_start(g0, 0)
            gather_start(g0 + 2, 0)
            gather_wait(1)
            write_wait(1)
            scale(1)
            write_start(g0 + 1, 1)
            return ()

        lax.fori_loop(1, n_chunks // 2 - 1, pair_body, ())

        g0 = n_chunks - 2
        gather_start(g0 + 1, 1)
        gather_wait(0)
        write_wait(0)
        scale(0)
        write_start(g0, 0)
        gather_wait(1)
        write_wait(1)
        scale(1)
        write_start(g0 + 1, 1)
        write_wait(0)
        write_wait(1)

    return emb_kernel


def kernel(x, lut):
    B = x.size
    x_2d = x.reshape(B // CHUNK, CHUNK).astype(jnp.int32)
    out = _build(B, lut.shape[0])(x_2d, lut)
    return out.reshape(*x.shape, D_MODEL)


# speedup vs baseline: 1.3760x; 1.1602x over previous
import functools
import math

import jax
import jax.numpy as jnp
from jax import lax
from jax.experimental import pallas as pl
from jax.experimental.pallas import tpu as pltpu
from jax.experimental.pallas import tpu_sc as plsc

D_MODEL = 64
SCALE = math.sqrt(D_MODEL)
CHUNK = 128


@functools.cache
def _build(B, V):
    info = plsc.get_sparse_core_info()
    nc, ns, lanes = info.num_cores, info.num_subcores, info.num_lanes
    nw = nc * ns
    b_per_w = B // nw
    n_chunks = b_per_w // CHUNK
    assert n_chunks % 2 == 0 and n_chunks >= 4
    mesh = plsc.VectorSubcoreMesh(core_axis_name="c", subcore_axis_name="s")

    @functools.partial(
        pl.kernel,
        mesh=mesh,
        compiler_params=pltpu.CompilerParams(needs_layout_passes=False),
        out_type=jax.ShapeDtypeStruct((B, D_MODEL), jnp.float32),
        scratch_types=[
            pltpu.VMEM((n_chunks, CHUNK), jnp.int32),
            pltpu.VMEM((n_chunks, CHUNK), jnp.int32),
            pltpu.VMEM((CHUNK, 2 * D_MODEL), jnp.float32),
            pltpu.VMEM((CHUNK, 2 * D_MODEL), jnp.float32),
            pltpu.VMEM((CHUNK, D_MODEL), jnp.float32),
            pltpu.VMEM((CHUNK, D_MODEL), jnp.float32),
            pltpu.SemaphoreType.DMA,
            pltpu.SemaphoreType.DMA,
            pltpu.SemaphoreType.DMA,
            pltpu.SemaphoreType.DMA,
        ],
    )
    def emb_kernel(p_hbm, h_hbm, lut_hbm, out_hbm, p_v, h_v, gbuf0, gbuf1,
                   wbuf0, wbuf1, gsem0, gsem1, wsem0, wsem1):
        wid = lax.axis_index("s") * nc + lax.axis_index("c")
        out_base = wid * b_per_w
        pltpu.sync_copy(p_hbm.at[pl.ds(wid * n_chunks, n_chunks)], p_v)
        pltpu.sync_copy(h_hbm.at[pl.ds(wid * n_chunks, n_chunks)], h_v)

        gbufs = (gbuf0, gbuf1)
        gsems = (gsem0, gsem1)
        wbufs = (wbuf0, wbuf1)
        wsems = (wsem0, wsem1)

        def gather_start(g, b):
            pltpu.async_copy(lut_hbm.at[p_v.at[g]], gbufs[b], gsems[b])

        def scale(g, b):
            src, dst = gbufs[b], wbufs[b]

            @plsc.parallel_loop(0, CHUNK, unroll=4)
            def _(r):
                gg = jnp.full((lanes,), g, jnp.int32)
                rr = jnp.full((lanes,), r, jnp.int32)
                hi = plsc.load_gather(h_v, [gg, rr]) != 0
                for j in range(D_MODEL // lanes):
                    lo_sl = pl.ds(j * lanes, lanes)
                    hi_sl = pl.ds(D_MODEL + j * lanes, lanes)
                    v = jnp.where(hi, src[r, hi_sl], src[r, lo_sl])
                    dst[r, lo_sl] = v * SCALE

        def write_start(g, b):
            pltpu.async_copy(wbufs[b],
                             out_hbm.at[pl.ds(out_base + g * CHUNK, CHUNK)],
                             wsems[b])

        def gather_wait(b):
            pltpu.make_async_copy(lut_hbm.at[p_v.at[0]], gbufs[b],
                                  gsems[b]).wait()

        def write_wait(b):
            pltpu.make_async_copy(wbufs[b], out_hbm.at[pl.ds(0, CHUNK)],
                                  wsems[b]).wait()

        gather_start(0, 0)
        gather_start(1, 1)
        gather_wait(0)
        scale(0, 0)
        write_start(0, 0)
        gather_start(2, 0)
        gather_wait(1)
        scale(1, 1)
        write_start(1, 1)

        def pair_body(t, _):
            g0 = 2 * t
            gather_start(g0 + 1, 1)
            gather_wait(0)
            write_wait(0)
            scale(g0, 0)
            write_start(g0, 0)
            gather_start(g0 + 2, 0)
            gather_wait(1)
            write_wait(1)
            scale(g0 + 1, 1)
            write_start(g0 + 1, 1)
            return ()

        lax.fori_loop(1, n_chunks // 2 - 1, pair_body, ())

        g0 = n_chunks - 2
        gather_start(g0 + 1, 1)
        gather_wait(0)
        write_wait(0)
        scale(g0, 0)
        write_start(g0, 0)
        gather_wait(1)
        write_wait(1)
        scale(g0 + 1, 1)
        write_start(g0 + 1, 1)
        write_wait(0)
        write_wait(1)

    return emb_kernel


def kernel(x, lut):
    B = x.size
    x2 = x.reshape(B // CHUNK, CHUNK).astype(jnp.int32)
    out = _build(B, lut.shape[0])(x2 >> 1, x2 & 1,
                                  lut.reshape(lut.shape[0] // 2, 2 * D_MODEL))
    return out.reshape(*x.shape, D_MODEL)
